# baseline (speedup 1.0000x reference)
import functools

import jax
import jax.numpy as jnp
from jax import lax
from jax.experimental import pallas as pl
from jax.experimental.pallas import tpu as pltpu
from jax.experimental.pallas import tpu_sc as plsc

VOCAB = 1000000
EMBED_DIM = 64
SEQ_LEN = 200
BATCH = 4096

NUM_CORES = 2
NUM_SUBCORES = 16
NUM_WORKERS = NUM_CORES * NUM_SUBCORES
BLK = BATCH // NUM_WORKERS
NBUF = 4
NSPLIT = 1

_mesh = plsc.VectorSubcoreMesh(core_axis_name="c", subcore_axis_name="s")


def _make_embed(nseq):
    assert nseq >= 2 * NBUF

    @functools.partial(
        pl.kernel,
        mesh=_mesh,
        out_type=jax.ShapeDtypeStruct((nseq, BATCH, EMBED_DIM), jnp.float32),
        scratch_types=[
            pltpu.VMEM((nseq, BLK), jnp.int32),
            [pltpu.VMEM((BLK, EMBED_DIM), jnp.float32) for _ in range(NBUF)],
            [pltpu.SemaphoreType.DMA for _ in range(NBUF)],
            [pltpu.SemaphoreType.DMA for _ in range(NBUF)],
            [pltpu.SemaphoreType.DMA for _ in range(NBUF)],
        ],
        compiler_params=pltpu.CompilerParams(use_tc_tiling_on_sc=False),
    )
    def _embed(ids_t_hbm, wt_hbm, pos_rep_hbm, out_hbm, idx_all, rows,
               gsem, ssem, isem):
        wid = lax.axis_index("s") * NUM_CORES + lax.axis_index("c")
        pltpu.sync_copy(ids_t_hbm.at[:, pl.ds(wid * BLK, BLK)], idx_all)

        def init_start(s, b):
            pltpu.async_copy(pos_rep_hbm.at[s], rows[b], isem[b])

        def init_wait(s, b):
            pltpu.make_async_copy(
                pos_rep_hbm.at[s], rows[b], isem[b]).wait()

        def gather_start(s, b):
            pltpu.async_copy(wt_hbm.at[idx_all.at[s]], rows[b], gsem[b],
                             add=True)

        def gather_wait(s, b):
            pltpu.make_async_copy(
                wt_hbm.at[idx_all.at[s]], rows[b], gsem[b]).wait()

        def out_slice(s):
            return out_hbm.at[s, pl.ds(wid * BLK, BLK)]

        def store_start(s, b):
            pltpu.async_copy(rows[b], out_slice(s), ssem[b])

        def store_wait(s, b):
            pltpu.make_async_copy(rows[b], out_slice(s), ssem[b]).wait()

        for k in range(3):
            init_start(k, k)
        for k in range(2):
            init_wait(k, k)
            gather_start(k, k)

        def step(s, b, head, tail3, tail2):
            gather_wait(s, b)
            store_start(s, b)
            b3 = (b + NBUF - 1) % NBUF
            b2 = (b + NBUF - 2) % NBUF
            if tail3:
                if not head:
                    store_wait(s - 1, b3)
                init_start(s + NBUF - 1, b3)
            if tail2:
                init_wait(s + NBUF - 2, b2)
                gather_start(s + NBUF - 2, b2)

        step(0, 0, True, True, True)

        def group_body(i, carry):
            for k in range(NBUF):
                step(NBUF * i + 1 + k, (1 + k) % NBUF, False, True, True)
            return carry

        n_full = nseq - NBUF
        lax.fori_loop(0, n_full // NBUF, group_body, 0)
        for r in range(n_full % NBUF):
            s = 1 + (n_full // NBUF) * NBUF + r
            step(s, s % NBUF, False, True, True)

        s1 = nseq - NBUF + 1
        step(s1, s1 % NBUF, False, False, True)
        for s in range(nseq - NBUF + 2, nseq):
            gather_wait(s, s % NBUF)
            store_start(s, s % NBUF)

        for s in range(nseq - NBUF, nseq):
            store_wait(s, s % NBUF)

    return _embed


_embed_chunk = _make_embed(SEQ_LEN // NSPLIT)


def kernel(input_ids, word_table, pos_table):
    ids_t = input_ids.T.astype(jnp.int32)
    pos_rep = jnp.broadcast_to(pos_table[:, None, :], (SEQ_LEN, BLK, EMBED_DIM))
    out_t = _embed_chunk(ids_t, word_table, pos_rep)
    return out_t.transpose(1, 0, 2)
